# baseline (speedup 1.0000x reference)
import functools

import jax
import jax.numpy as jnp
import numpy as np
from jax import lax
from jax.experimental import pallas as pl
from jax.experimental.pallas import tpu as pltpu
from jax.experimental.pallas import tpu_sc as plsc

N_NODES = 10000
D_FEAT = 128
N_EDGES = 320000

NUM_CORES = 2
NUM_SUBCORES = 16
NUM_WORKERS = NUM_CORES * NUM_SUBCORES
EDGES_PER_WORKER = N_EDGES // NUM_WORKERS
CHUNK = 80
NUM_CHUNKS = EDGES_PER_WORKER // CHUNK
UNROLL = 4

def _hsum16(v):
    lane = lax.iota(jnp.int32, 16)
    for s in (8, 4, 2, 1):
        perm = lax.bitwise_xor(lane, jnp.int32(s))
        v = v + v.at[perm].get(mode="promise_in_bounds")
    return v


def _sqrt_newton(x):
    xs = jnp.maximum(x, jnp.float32(1e-30))
    i = lax.bitcast_convert_type(xs, jnp.int32)
    i = jnp.int32(0x5F3759DF) - lax.shift_right_arithmetic(i, jnp.int32(1))
    y = lax.bitcast_convert_type(i, jnp.float32)
    for _ in range(3):
        y = y * (jnp.float32(1.5) - jnp.float32(0.5) * xs * y * y)
    return xs * y


def _body(embf, embw, src, tgt, td, cf, out,
          src_v, tgt_v, td_v, cf_v,
          srow_a, trow_a, srow_b, trow_b, res_v, xq_v,
          sem_a, sem_b):
    wid = lax.axis_index("s") * NUM_CORES + lax.axis_index("c")
    base = wid * EDGES_PER_WORKER

    pltpu.sync_copy(src.at[pl.ds(base, EDGES_PER_WORKER)], src_v)
    pltpu.sync_copy(tgt.at[pl.ds(base, EDGES_PER_WORKER)], tgt_v)
    pltpu.sync_copy(td.at[pl.ds(base, EDGES_PER_WORKER)], td_v)
    pltpu.sync_copy(cf.at[pl.ds(base, EDGES_PER_WORKER)], cf_v)

    lane = lax.iota(jnp.int32, 16)
    bufs = ((srow_a, trow_a, sem_a), (srow_b, trow_b, sem_b))

    def fire(c, which):
        srow, trow, sem = bufs[which]
        off = c * CHUNK
        pltpu.async_copy(embf.at[src_v.at[pl.ds(off, CHUNK)]], srow, sem)
        pltpu.async_copy(embw.at[tgt_v.at[pl.ds(off, CHUNK)]], trow, sem)

    def compute(c, which, acc):
        srow, trow, sem = bufs[which]
        pltpu.make_async_copy(embf.at[src_v.at[pl.ds(0, CHUNK)]], srow, sem).wait()
        pltpu.make_async_copy(embw.at[tgt_v.at[pl.ds(0, CHUNK)]], trow, sem).wait()
        off = c * CHUNK

        himask = jnp.full((16,), 0xFFFF0000, jnp.uint32).astype(jnp.int32)
        perm_e = lax.bitwise_and(lane * 2, jnp.int32(15))
        perm_o = lax.bitwise_and(lane * 2 + 1, jnp.int32(15))
        mlow = lane < 8

        def split(w):
            hi = lax.bitcast_convert_type(
                lax.bitwise_and(w, himask), jnp.float32)
            lo = lax.bitcast_convert_type(
                lax.shift_left(w, jnp.int32(16)), jnp.float32)
            return lo, hi

        def hadd(a, b):
            u = jnp.where(mlow, a.at[perm_e].get(mode="promise_in_bounds"),
                          b.at[perm_e].get(mode="promise_in_bounds"))
            w = jnp.where(mlow, a.at[perm_o].get(mode="promise_in_bounds"),
                          b.at[perm_o].get(mode="promise_in_bounds"))
            return u + w

        def group_body(g, a16):
            gbase = g * 16

            def quad_body(q, _):
                vecs = []
                for u in range(UNROLL):
                    e = gbase + q * UNROLL + u
                    s16 = None
                    for k in range(4):
                        t0, t1 = split(trow[e, pl.ds(k * 16, 16)])
                        s0 = srow[e, pl.ds(k * 16, 16)]
                        s1 = srow[e, pl.ds(64 + k * 16, 16)]
                        d0 = s0 - t0
                        d1 = s1 - t1
                        if s16 is None:
                            s16 = d0 * d0
                        else:
                            s16 = s16 + d0 * d0
                        s16 = s16 + d1 * d1
                    vecs.append(s16)
                xq_v[q, :] = hadd(hadd(vecs[0], vecs[1]),
                                  hadd(vecs[2], vecs[3]))
                return 0

            lax.fori_loop(0, 16 // UNROLL, quad_body, 0)
            sumsq = hadd(hadd(xq_v[0, :], xq_v[1, :]),
                         hadd(xq_v[2, :], xq_v[3, :]))
            dist = _sqrt_newton(sumsq)
            err = dist - td_v[pl.ds(off + gbase, 16)]
            return a16 + err * err * cf_v[pl.ds(off + gbase, 16)]

        return lax.fori_loop(0, CHUNK // 16, group_body, acc)

    fire(0, 0)

    def pair_body(i, acc):
        c0 = 2 * i
        fire(c0 + 1, 1)
        acc = compute(c0, 0, acc)
        fire(c0 + 2, 0)
        return compute(c0 + 1, 1, acc)

    acc16 = lax.fori_loop(0, (NUM_CHUNKS - 1) // 2, pair_body,
                          jnp.zeros((16,), jnp.float32))
    acc16 = compute(NUM_CHUNKS - 1, 0, acc16)

    total = _hsum16(acc16) * jnp.float32(1.0 / N_EDGES)
    res_v[...] = jnp.where(lane == 0, total, jnp.float32(0.0))
    pltpu.sync_copy(res_v, out.at[wid])


def kernel(embeddings, source_id, target_id, target_distance, confidence):
    emb16 = embeddings.astype(jnp.bfloat16)
    embw = lax.bitcast_convert_type(
        jnp.stack([emb16[:, :64], emb16[:, 64:]], axis=-1), jnp.int32)
    mesh = plsc.VectorSubcoreMesh(core_axis_name="c", subcore_axis_name="s")
    f = pl.kernel(
        _body,
        mesh=mesh,
        out_type=jax.ShapeDtypeStruct((NUM_WORKERS, 16), jnp.float32),
        compiler_params=pltpu.CompilerParams(use_tc_tiling_on_sc=False),
        scratch_types=[
            pltpu.VMEM((EDGES_PER_WORKER,), jnp.int32),
            pltpu.VMEM((EDGES_PER_WORKER,), jnp.int32),
            pltpu.VMEM((EDGES_PER_WORKER,), jnp.float32),
            pltpu.VMEM((EDGES_PER_WORKER,), jnp.float32),
            pltpu.VMEM((CHUNK, D_FEAT), jnp.float32),
            pltpu.VMEM((CHUNK, D_FEAT // 2), jnp.int32),
            pltpu.VMEM((CHUNK, D_FEAT), jnp.float32),
            pltpu.VMEM((CHUNK, D_FEAT // 2), jnp.int32),
            pltpu.VMEM((16,), jnp.float32),
            pltpu.VMEM((4, 16), jnp.float32),
            pltpu.SemaphoreType.DMA,
            pltpu.SemaphoreType.DMA,
        ],
    )
    partials = f(embeddings, embw, source_id, target_id,
                 target_distance, confidence)
    return jnp.sum(partials)


# speedup vs baseline: 1.0475x; 1.0475x over previous
import functools

import jax
import jax.numpy as jnp
import numpy as np
from jax import lax
from jax.experimental import pallas as pl
from jax.experimental.pallas import tpu as pltpu
from jax.experimental.pallas import tpu_sc as plsc

N_NODES = 10000
D_FEAT = 128
N_EDGES = 320000

NUM_CORES = 2
NUM_SUBCORES = 16
NUM_WORKERS = NUM_CORES * NUM_SUBCORES
EDGES_PER_WORKER = N_EDGES // NUM_WORKERS
CHUNK = 80
NUM_CHUNKS = EDGES_PER_WORKER // CHUNK
UNROLL = 8

def _hsum16(v):
    lane = lax.iota(jnp.int32, 16)
    for s in (8, 4, 2, 1):
        perm = lax.bitwise_xor(lane, jnp.int32(s))
        v = v + v.at[perm].get(mode="promise_in_bounds")
    return v


def _sqrt_newton(x):
    xs = jnp.maximum(x, jnp.float32(1e-30))
    i = lax.bitcast_convert_type(xs, jnp.int32)
    i = jnp.int32(0x5F3759DF) - lax.shift_right_arithmetic(i, jnp.int32(1))
    y = lax.bitcast_convert_type(i, jnp.float32)
    for _ in range(3):
        y = y * (jnp.float32(1.5) - jnp.float32(0.5) * xs * y * y)
    return xs * y


def _body(embf, embw, src, tgt, td, cf, out,
          src_v, tgt_v, td_v, cf_v,
          srow_a, trow_a, srow_b, trow_b, res_v,
          sem_a, sem_b):
    wid = lax.axis_index("s") * NUM_CORES + lax.axis_index("c")
    base = wid * EDGES_PER_WORKER

    pltpu.sync_copy(src.at[pl.ds(base, EDGES_PER_WORKER)], src_v)
    pltpu.sync_copy(tgt.at[pl.ds(base, EDGES_PER_WORKER)], tgt_v)
    pltpu.sync_copy(td.at[pl.ds(base, EDGES_PER_WORKER)], td_v)
    pltpu.sync_copy(cf.at[pl.ds(base, EDGES_PER_WORKER)], cf_v)

    lane = lax.iota(jnp.int32, 16)
    bufs = ((srow_a, trow_a, sem_a), (srow_b, trow_b, sem_b))

    def fire(c, which):
        srow, trow, sem = bufs[which]
        off = c * CHUNK
        pltpu.async_copy(embf.at[src_v.at[pl.ds(off, CHUNK)]], srow, sem)
        pltpu.async_copy(embw.at[tgt_v.at[pl.ds(off, CHUNK)]], trow, sem)

    def compute(c, which, acc):
        srow, trow, sem = bufs[which]
        pltpu.make_async_copy(embf.at[src_v.at[pl.ds(0, CHUNK)]], srow, sem).wait()
        pltpu.make_async_copy(embw.at[tgt_v.at[pl.ds(0, CHUNK)]], trow, sem).wait()
        off = c * CHUNK

        def group_body(g, a16):
            gbase = g * 16
            himask = jnp.full((16,), 0xFFFF0000, jnp.uint32).astype(jnp.int32)

            def split(w):
                hi = lax.bitcast_convert_type(
                    lax.bitwise_and(w, himask), jnp.float32)
                lo = lax.bitcast_convert_type(
                    lax.shift_left(w, jnp.int32(16)), jnp.float32)
                return lo, hi

            def quad_body(q, sumsq):
                for u in range(UNROLL):
                    j = q * UNROLL + u
                    e = gbase + j
                    s16 = None
                    for k in range(4):
                        t0, t1 = split(trow[e, pl.ds(k * 16, 16)])
                        s0 = srow[e, pl.ds(k * 16, 16)]
                        s1 = srow[e, pl.ds(64 + k * 16, 16)]
                        d0 = s0 - t0
                        d1 = s1 - t1
                        if s16 is None:
                            s16 = d0 * d0
                        else:
                            s16 = s16 + d0 * d0
                        s16 = s16 + d1 * d1
                    sumsq = jnp.where(lane == j, _hsum16(s16), sumsq)
                return sumsq

            sumsq = lax.fori_loop(0, 16 // UNROLL, quad_body,
                                  jnp.zeros((16,), jnp.float32))
            dist = _sqrt_newton(sumsq)
            err = dist - td_v[pl.ds(off + gbase, 16)]
            return a16 + err * err * cf_v[pl.ds(off + gbase, 16)]

        return lax.fori_loop(0, CHUNK // 16, group_body, acc)

    fire(0, 0)

    def pair_body(i, acc):
        c0 = 2 * i
        fire(c0 + 1, 1)
        acc = compute(c0, 0, acc)
        fire(c0 + 2, 0)
        return compute(c0 + 1, 1, acc)

    acc16 = lax.fori_loop(0, (NUM_CHUNKS - 1) // 2, pair_body,
                          jnp.zeros((16,), jnp.float32))
    acc16 = compute(NUM_CHUNKS - 1, 0, acc16)

    total = _hsum16(acc16) * jnp.float32(1.0 / N_EDGES)
    res_v[...] = jnp.where(lane == 0, total, jnp.float32(0.0))
    pltpu.sync_copy(res_v, out.at[wid])


def kernel(embeddings, source_id, target_id, target_distance, confidence):
    emb16 = embeddings.astype(jnp.bfloat16)
    embw = lax.bitcast_convert_type(
        jnp.stack([emb16[:, :64], emb16[:, 64:]], axis=-1), jnp.int32)
    mesh = plsc.VectorSubcoreMesh(core_axis_name="c", subcore_axis_name="s")
    f = pl.kernel(
        _body,
        mesh=mesh,
        out_type=jax.ShapeDtypeStruct((NUM_WORKERS, 16), jnp.float32),
        compiler_params=pltpu.CompilerParams(use_tc_tiling_on_sc=False),
        scratch_types=[
            pltpu.VMEM((EDGES_PER_WORKER,), jnp.int32),
            pltpu.VMEM((EDGES_PER_WORKER,), jnp.int32),
            pltpu.VMEM((EDGES_PER_WORKER,), jnp.float32),
            pltpu.VMEM((EDGES_PER_WORKER,), jnp.float32),
            pltpu.VMEM((CHUNK, D_FEAT), jnp.float32),
            pltpu.VMEM((CHUNK, D_FEAT // 2), jnp.int32),
            pltpu.VMEM((CHUNK, D_FEAT), jnp.float32),
            pltpu.VMEM((CHUNK, D_FEAT // 2), jnp.int32),
            pltpu.VMEM((16,), jnp.float32),
            pltpu.SemaphoreType.DMA,
            pltpu.SemaphoreType.DMA,
        ],
    )
    partials = f(embeddings, embw, source_id, target_id,
                 target_distance, confidence)
    return jnp.sum(partials)


# speedup vs baseline: 1.0816x; 1.0326x over previous
import functools

import jax
import jax.numpy as jnp
import numpy as np
from jax import lax
from jax.experimental import pallas as pl
from jax.experimental.pallas import tpu as pltpu
from jax.experimental.pallas import tpu_sc as plsc

N_NODES = 10000
D_FEAT = 128
N_EDGES = 320000

NUM_CORES = 2
NUM_SUBCORES = 16
NUM_WORKERS = NUM_CORES * NUM_SUBCORES
EDGES_PER_WORKER = N_EDGES // NUM_WORKERS
CHUNK = 80
NUM_CHUNKS = EDGES_PER_WORKER // CHUNK
UNROLL = 4

def _hsum16(v):
    lane = lax.iota(jnp.int32, 16)
    for s in (8, 4, 2, 1):
        perm = lax.bitwise_xor(lane, jnp.int32(s))
        v = v + v.at[perm].get(mode="promise_in_bounds")
    return v


def _sqrt_newton(x):
    xs = jnp.maximum(x, jnp.float32(1e-30))
    i = lax.bitcast_convert_type(xs, jnp.int32)
    i = jnp.int32(0x5F3759DF) - lax.shift_right_arithmetic(i, jnp.int32(1))
    y = lax.bitcast_convert_type(i, jnp.float32)
    for _ in range(3):
        y = y * (jnp.float32(1.5) - jnp.float32(0.5) * xs * y * y)
    return xs * y


def _body(embf, embw, src, tgt, td, cf, out,
          src_v, tgt_v, td_v, cf_v,
          srow_a, trow_a, srow_b, trow_b, res_v,
          sem_a, sem_b):
    wid = lax.axis_index("s") * NUM_CORES + lax.axis_index("c")
    base = wid * EDGES_PER_WORKER

    pltpu.sync_copy(src.at[pl.ds(base, EDGES_PER_WORKER)], src_v)
    pltpu.sync_copy(tgt.at[pl.ds(base, EDGES_PER_WORKER)], tgt_v)
    pltpu.sync_copy(td.at[pl.ds(base, EDGES_PER_WORKER)], td_v)
    pltpu.sync_copy(cf.at[pl.ds(base, EDGES_PER_WORKER)], cf_v)

    lane = lax.iota(jnp.int32, 16)
    bufs = ((srow_a, trow_a, sem_a), (srow_b, trow_b, sem_b))

    def fire(c, which):
        srow, trow, sem = bufs[which]
        off = c * CHUNK
        pltpu.async_copy(embf.at[src_v.at[pl.ds(off, CHUNK)]], srow, sem)
        pltpu.async_copy(embw.at[tgt_v.at[pl.ds(off, CHUNK)]], trow, sem)

    def compute(c, which, acc):
        srow, trow, sem = bufs[which]
        pltpu.make_async_copy(embf.at[src_v.at[pl.ds(0, CHUNK)]], srow, sem).wait()
        pltpu.make_async_copy(embw.at[tgt_v.at[pl.ds(0, CHUNK)]], trow, sem).wait()
        off = c * CHUNK

        def group_body(g, a16):
            gbase = g * 16
            himask = jnp.full((16,), 0xFFFF0000, jnp.uint32).astype(jnp.int32)

            def split(w):
                hi = lax.bitcast_convert_type(
                    lax.bitwise_and(w, himask), jnp.float32)
                lo = lax.bitcast_convert_type(
                    lax.shift_left(w, jnp.int32(16)), jnp.float32)
                return lo, hi

            def quad_body(q, sumsq):
                for u in range(UNROLL):
                    j = q * UNROLL + u
                    e = gbase + j
                    s16 = None
                    for k in range(4):
                        t0, t1 = split(trow[e, pl.ds(k * 16, 16)])
                        s0 = srow[e, pl.ds(k * 16, 16)]
                        s1 = srow[e, pl.ds(64 + k * 16, 16)]
                        d0 = s0 - t0
                        d1 = s1 - t1
                        if s16 is None:
                            s16 = d0 * d0
                        else:
                            s16 = s16 + d0 * d0
                        s16 = s16 + d1 * d1
                    sumsq = jnp.where(lane == j, _hsum16(s16), sumsq)
                return sumsq

            sumsq = lax.fori_loop(0, 16 // UNROLL, quad_body,
                                  jnp.zeros((16,), jnp.float32))
            dist = _sqrt_newton(sumsq)
            err = dist - td_v[pl.ds(off + gbase, 16)]
            return a16 + err * err * cf_v[pl.ds(off + gbase, 16)]

        return lax.fori_loop(0, CHUNK // 16, group_body, acc)

    fire(0, 0)

    def pair_body(i, acc):
        c0 = 2 * i
        fire(c0 + 1, 1)
        acc = compute(c0, 0, acc)
        fire(c0 + 2, 0)
        return compute(c0 + 1, 1, acc)

    acc16 = lax.fori_loop(0, (NUM_CHUNKS - 1) // 2, pair_body,
                          jnp.zeros((16,), jnp.float32))
    acc16 = compute(NUM_CHUNKS - 1, 0, acc16)

    total = _hsum16(acc16) * jnp.float32(1.0 / N_EDGES)
    res_v[...] = jnp.where(lane == 0, total, jnp.float32(0.0))
    pltpu.sync_copy(res_v, out.at[wid])


def kernel(embeddings, source_id, target_id, target_distance, confidence):
    v = lax.bitcast_convert_type(embeddings, jnp.uint32)
    r = v + jnp.uint32(0x7FFF) + ((v >> jnp.uint32(16)) & jnp.uint32(1))
    lo = r[:, :64] >> jnp.uint32(16)
    hi = r[:, 64:] & jnp.uint32(0xFFFF0000)
    embw = lax.bitcast_convert_type(lo | hi, jnp.int32)
    mesh = plsc.VectorSubcoreMesh(core_axis_name="c", subcore_axis_name="s")
    f = pl.kernel(
        _body,
        mesh=mesh,
        out_type=jax.ShapeDtypeStruct((NUM_WORKERS, 16), jnp.float32),
        compiler_params=pltpu.CompilerParams(use_tc_tiling_on_sc=False),
        scratch_types=[
            pltpu.VMEM((EDGES_PER_WORKER,), jnp.int32),
            pltpu.VMEM((EDGES_PER_WORKER,), jnp.int32),
            pltpu.VMEM((EDGES_PER_WORKER,), jnp.float32),
            pltpu.VMEM((EDGES_PER_WORKER,), jnp.float32),
            pltpu.VMEM((CHUNK, D_FEAT), jnp.float32),
            pltpu.VMEM((CHUNK, D_FEAT // 2), jnp.int32),
            pltpu.VMEM((CHUNK, D_FEAT), jnp.float32),
            pltpu.VMEM((CHUNK, D_FEAT // 2), jnp.int32),
            pltpu.VMEM((16,), jnp.float32),
            pltpu.SemaphoreType.DMA,
            pltpu.SemaphoreType.DMA,
        ],
    )
    partials = f(embeddings, embw, source_id, target_id,
                 target_distance, confidence)
    return jnp.sum(partials)


# speedup vs baseline: 1.3120x; 1.2129x over previous
import functools

import jax
import jax.numpy as jnp
import numpy as np
from jax import lax
from jax.experimental import pallas as pl
from jax.experimental.pallas import tpu as pltpu
from jax.experimental.pallas import tpu_sc as plsc

N_NODES = 10000
D_FEAT = 128
N_EDGES = 320000

NUM_CORES = 2
NUM_SUBCORES = 16
NUM_WORKERS = NUM_CORES * NUM_SUBCORES
EDGES_PER_WORKER = N_EDGES // NUM_WORKERS
CHUNK = 80
NUM_CHUNKS = EDGES_PER_WORKER // CHUNK
UNROLL = 4

def _hsum16(v):
    lane = lax.iota(jnp.int32, 16)
    for s in (8, 4, 2, 1):
        perm = lax.bitwise_xor(lane, jnp.int32(s))
        v = v + v.at[perm].get(mode="promise_in_bounds")
    return v


def _sqrt_newton(x):
    xs = jnp.maximum(x, jnp.float32(1e-30))
    i = lax.bitcast_convert_type(xs, jnp.int32)
    i = jnp.int32(0x5F3759DF) - lax.shift_right_arithmetic(i, jnp.int32(1))
    y = lax.bitcast_convert_type(i, jnp.float32)
    for _ in range(3):
        y = y * (jnp.float32(1.5) - jnp.float32(0.5) * xs * y * y)
    return xs * y


def _body(embf, embw, src, tgt, td, cf, out,
          src_v, tgt_v, td_v, cf_v,
          srow_a, trow_a, srow_b, trow_b, srow_c, trow_c, res_v,
          sem_a, sem_b, sem_c):
    wid = lax.axis_index("s") * NUM_CORES + lax.axis_index("c")
    base = wid * EDGES_PER_WORKER

    pltpu.sync_copy(src.at[pl.ds(base, EDGES_PER_WORKER)], src_v)
    pltpu.sync_copy(tgt.at[pl.ds(base, EDGES_PER_WORKER)], tgt_v)
    pltpu.sync_copy(td.at[pl.ds(base, EDGES_PER_WORKER)], td_v)
    pltpu.sync_copy(cf.at[pl.ds(base, EDGES_PER_WORKER)], cf_v)

    lane = lax.iota(jnp.int32, 16)
    bufs = ((srow_a, trow_a, sem_a), (srow_b, trow_b, sem_b),
            (srow_c, trow_c, sem_c))

    def fire(c, which):
        srow, trow, sem = bufs[which]
        off = c * CHUNK
        pltpu.async_copy(embf.at[src_v.at[pl.ds(off, CHUNK)]], srow, sem)
        pltpu.async_copy(embw.at[tgt_v.at[pl.ds(off, CHUNK)]], trow, sem)

    def compute(c, which, acc):
        srow, trow, sem = bufs[which]
        pltpu.make_async_copy(embf.at[src_v.at[pl.ds(0, CHUNK)]], srow, sem).wait()
        pltpu.make_async_copy(embw.at[tgt_v.at[pl.ds(0, CHUNK)]], trow, sem).wait()
        off = c * CHUNK

        def group_body(g, a16):
            gbase = g * 16
            himask = jnp.full((16,), 0xFFFF0000, jnp.uint32).astype(jnp.int32)

            def split(w):
                hi = lax.bitcast_convert_type(
                    lax.bitwise_and(w, himask), jnp.float32)
                lo = lax.bitcast_convert_type(
                    lax.shift_left(w, jnp.int32(16)), jnp.float32)
                return lo, hi

            def quad_body(q, sumsq):
                for u in range(UNROLL):
                    j = q * UNROLL + u
                    e = gbase + j
                    s16 = None
                    for k in range(4):
                        t0, t1 = split(trow[e, pl.ds(k * 16, 16)])
                        s0 = srow[e, pl.ds(k * 16, 16)]
                        s1 = srow[e, pl.ds(64 + k * 16, 16)]
                        d0 = s0 - t0
                        d1 = s1 - t1
                        if s16 is None:
                            s16 = d0 * d0
                        else:
                            s16 = s16 + d0 * d0
                        s16 = s16 + d1 * d1
                    sumsq = jnp.where(lane == j, _hsum16(s16), sumsq)
                return sumsq

            sumsq = lax.fori_loop(0, 16 // UNROLL, quad_body,
                                  jnp.zeros((16,), jnp.float32))
            dist = _sqrt_newton(sumsq)
            err = dist - td_v[pl.ds(off + gbase, 16)]
            return a16 + err * err * cf_v[pl.ds(off + gbase, 16)]

        return lax.fori_loop(0, CHUNK // 16, group_body, acc)

    fire(0, 0)
    fire(1, 1)

    def triple_body(i, acc):
        c = 3 * i
        fire(c + 2, 2)
        acc = compute(c, 0, acc)
        fire(c + 3, 0)
        acc = compute(c + 1, 1, acc)
        fire(c + 4, 1)
        return compute(c + 2, 2, acc)

    acc16 = lax.fori_loop(0, (NUM_CHUNKS - 2) // 3, triple_body,
                          jnp.zeros((16,), jnp.float32))
    acc16 = compute(NUM_CHUNKS - 2, 0, acc16)
    acc16 = compute(NUM_CHUNKS - 1, 1, acc16)

    total = _hsum16(acc16) * jnp.float32(1.0 / N_EDGES)
    res_v[...] = jnp.where(lane == 0, total, jnp.float32(0.0))
    pltpu.sync_copy(res_v, out.at[wid])


def kernel(embeddings, source_id, target_id, target_distance, confidence):
    v = lax.bitcast_convert_type(embeddings, jnp.uint32)
    r = v + jnp.uint32(0x7FFF) + ((v >> jnp.uint32(16)) & jnp.uint32(1))
    lo = r[:, :64] >> jnp.uint32(16)
    hi = r[:, 64:] & jnp.uint32(0xFFFF0000)
    embw = lax.bitcast_convert_type(lo | hi, jnp.int32)
    mesh = plsc.VectorSubcoreMesh(core_axis_name="c", subcore_axis_name="s")
    f = pl.kernel(
        _body,
        mesh=mesh,
        out_type=jax.ShapeDtypeStruct((NUM_WORKERS, 16), jnp.float32),
        compiler_params=pltpu.CompilerParams(use_tc_tiling_on_sc=False),
        scratch_types=[
            pltpu.VMEM((EDGES_PER_WORKER,), jnp.int32),
            pltpu.VMEM((EDGES_PER_WORKER,), jnp.int32),
            pltpu.VMEM((EDGES_PER_WORKER,), jnp.float32),
            pltpu.VMEM((EDGES_PER_WORKER,), jnp.float32),
            pltpu.VMEM((CHUNK, D_FEAT), jnp.float32),
            pltpu.VMEM((CHUNK, D_FEAT // 2), jnp.int32),
            pltpu.VMEM((CHUNK, D_FEAT), jnp.float32),
            pltpu.VMEM((CHUNK, D_FEAT // 2), jnp.int32),
            pltpu.VMEM((CHUNK, D_FEAT), jnp.float32),
            pltpu.VMEM((CHUNK, D_FEAT // 2), jnp.int32),
            pltpu.VMEM((16,), jnp.float32),
            pltpu.SemaphoreType.DMA,
            pltpu.SemaphoreType.DMA,
            pltpu.SemaphoreType.DMA,
        ],
    )
    partials = f(embeddings, embw, source_id, target_id,
                 target_distance, confidence)
    return jnp.sum(partials)
